# baseline (speedup 1.0000x reference)
import functools

import jax
import jax.numpy as jnp
from jax import lax
from jax.experimental import pallas as pl
from jax.experimental.pallas import tpu as pltpu
from jax.experimental.pallas import tpu_sc as plsc

_NC = 2
_NS = 16
_NW = _NC * _NS
_RANK = 128
_ACC_ROWS = 10112
_CHUNK = 128
_CCHUNK = 80
_R = 7
_D = 5
_SR = 3
_SD = _SR - 2
_SG = 1
_SLOPE = (1.0 / 8.0 + 1.0 / 3.0) / 2.0


def _scatter_sum(table, idx, dst, zeros):
    ne = idx.shape[0]
    e_pt = ne // _NW
    nch = e_pt // _CHUNK
    ngroups = nch // _SG
    assert e_pt * _NW == ne and ngroups * _SG * _CHUNK == e_pt
    assert ngroups >= _SR
    rpt = _ACC_ROWS // _NS
    assert zeros.shape[0] == rpt

    mesh = plsc.VectorSubcoreMesh(core_axis_name="c", subcore_axis_name="s")

    nb = _SR * _SG
    scratch = ([pltpu.VMEM((_CHUNK, _RANK), jnp.float32)] * nb
               + [pltpu.VMEM((_CHUNK,), jnp.int32)] * (2 * nb)
               + [pltpu.VMEM_SHARED((_ACC_ROWS, _RANK), jnp.float32)]
               + [pltpu.SemaphoreType.DMA] * (3 * _SR))

    @functools.partial(
        pl.kernel,
        out_type=jax.ShapeDtypeStruct((_NC, _ACC_ROWS, _RANK), jnp.float32),
        mesh=mesh,
        scratch_types=scratch,
    )
    def k(table_hbm, idx_hbm, dst_hbm, z_hbm, out_hbm, *refs):
        rows = [refs[r * _SG:(r + 1) * _SG] for r in range(_SR)]
        idxb = [refs[nb + r * _SG:nb + (r + 1) * _SG] for r in range(_SR)]
        dstb = [refs[2 * nb + r * _SG:2 * nb + (r + 1) * _SG]
                for r in range(_SR)]
        acc = refs[3 * nb]
        sem_i = refs[3 * nb + 1:3 * nb + 1 + _SR]
        sem_g = refs[3 * nb + 1 + _SR:3 * nb + 1 + 2 * _SR]
        sem_s = refs[3 * nb + 1 + 2 * _SR:3 * nb + 1 + 3 * _SR]

        c = lax.axis_index("c")
        s = lax.axis_index("s")
        w = c * _NS + s
        eb = w * e_pt

        pltpu.sync_copy(z_hbm, acc.at[pl.ds(s * rpt, rpt), :])
        plsc.subcore_barrier()

        for g in (0, 1):
            for j in range(_SG):
                b = eb + (g * _SG + j) * _CHUNK
                pltpu.sync_copy(idx_hbm.at[pl.ds(b, _CHUNK)], idxb[g][j])
                pltpu.sync_copy(dst_hbm.at[pl.ds(b, _CHUNK)], dstb[g][j])
        for j in range(_SG):
            pltpu.async_copy(table_hbm.at[idxb[0][j]], rows[0][j], sem_g[0])

        def group(t, r):
            sc = (r + 2) % _SR
            g1 = (r + 1) % _SR

            @pl.when(t >= _SD)
            def _():
                for j in range(_SG):
                    pltpu.make_async_copy(rows[sc][j], acc.at[dstb[sc][j]],
                                          sem_s[sc]).wait()
            @pl.when(t + 2 < ngroups)
            def _():
                for j in range(_SG):
                    b2 = eb + ((t + 2) * _SG + j) * _CHUNK
                    pltpu.async_copy(idx_hbm.at[pl.ds(b2, _CHUNK)],
                                     idxb[sc][j], sem_i[sc])
                    pltpu.async_copy(dst_hbm.at[pl.ds(b2, _CHUNK)],
                                     dstb[sc][j], sem_i[sc])
            @pl.when(t + 1 < ngroups)
            def _():
                @pl.when(t >= 1)
                def _():
                    for j in range(_SG):
                        b1 = eb + ((t + 1) * _SG + j) * _CHUNK
                        pltpu.make_async_copy(idx_hbm.at[pl.ds(b1, _CHUNK)],
                                              idxb[g1][j], sem_i[g1]).wait()
                        pltpu.make_async_copy(dst_hbm.at[pl.ds(b1, _CHUNK)],
                                              dstb[g1][j], sem_i[g1]).wait()
                for j in range(_SG):
                    pltpu.async_copy(table_hbm.at[idxb[g1][j]], rows[g1][j],
                                     sem_g[g1])
            for j in range(_SG):
                pltpu.make_async_copy(table_hbm.at[idxb[r][j]], rows[r][j],
                                      sem_g[r]).wait()
            for j in range(_SG):
                pltpu.async_copy(rows[r][j], acc.at[dstb[r][j]], sem_s[r],
                                 add=True)

        def body(t, carry):
            for r in range(_SR):
                @pl.when(lax.rem(t, _SR) == r)
                def _(r=r):
                    group(t, r)
            return carry

        lax.fori_loop(0, ngroups, body, 0)
        for t in range(ngroups - _SD, ngroups):
            r = t % _SR
            for j in range(_SG):
                pltpu.make_async_copy(rows[r][j], acc.at[dstb[r][j]],
                                      sem_s[r]).wait()

        plsc.subcore_barrier()
        pltpu.sync_copy(acc.at[pl.ds(s * rpt, rpt), :],
                        out_hbm.at[c, pl.ds(s * rpt, rpt), :])

    return k(table, idx, dst, zeros)


def _count(flat, n_rows, zeros_c, ones):
    ne = flat.shape[0]
    e_pt = ne // _NW
    ngroups = e_pt // _CCHUNK
    assert ngroups * _CCHUNK == e_pt and ngroups >= _R
    rpt = n_rows // _NS
    assert rpt * _NS == n_rows and rpt % 128 == 0
    assert zeros_c.shape[0] == rpt

    mesh = plsc.VectorSubcoreMesh(core_axis_name="c", subcore_axis_name="s")

    scratch = ([pltpu.VMEM((_CCHUNK,), jnp.int32)] * _R
               + [pltpu.VMEM((_CCHUNK,), jnp.float32)]
               + [pltpu.VMEM_SHARED((n_rows,), jnp.float32)]
               + [pltpu.SemaphoreType.DMA] * (2 * _R))

    @functools.partial(
        pl.kernel,
        out_type=[jax.ShapeDtypeStruct((n_rows,), jnp.float32),
                  jax.ShapeDtypeStruct((n_rows,), jnp.float32)],
        mesh=mesh,
        scratch_types=scratch,
    )
    def k(flat_hbm, z_hbm, ones_hbm, out0_hbm, out1_hbm, *refs):
        idxb = refs[0:_R]
        ones_v = refs[_R]
        acc = refs[_R + 1]
        sem_i = refs[_R + 2:2 * _R + 2]
        sem_s = refs[2 * _R + 2:3 * _R + 2]
        c = lax.axis_index("c")
        s = lax.axis_index("s")
        w = c * _NS + s
        eb = w * e_pt

        pltpu.sync_copy(z_hbm, acc.at[pl.ds(s * rpt, rpt)])
        pltpu.sync_copy(ones_hbm, ones_v)
        plsc.subcore_barrier()

        for g in (0, 1):
            pltpu.sync_copy(flat_hbm.at[pl.ds(eb + g * _CCHUNK, _CCHUNK)],
                            idxb[g])

        def group(t, r):
            sc = (r + 2) % _R

            @pl.when(t >= _D)
            def _():
                pltpu.make_async_copy(ones_v, acc.at[idxb[sc]],
                                      sem_s[sc]).wait()

            @pl.when(t + 2 < ngroups)
            def _():
                pltpu.async_copy(
                    flat_hbm.at[pl.ds(eb + (t + 2) * _CCHUNK, _CCHUNK)],
                    idxb[sc], sem_i[sc])

            @pl.when(t >= 2)
            def _():
                pltpu.make_async_copy(
                    flat_hbm.at[pl.ds(eb + t * _CCHUNK, _CCHUNK)],
                    idxb[r], sem_i[r]).wait()

            pltpu.async_copy(ones_v, acc.at[idxb[r]], sem_s[r], add=True)

        def body(t, carry):
            for r in range(_R):
                @pl.when(lax.rem(t, _R) == r)
                def _(r=r):
                    group(t, r)
            return carry

        lax.fori_loop(0, ngroups, body, 0)
        for t in range(ngroups - _D, ngroups):
            r = t % _R
            pltpu.make_async_copy(ones_v, acc.at[idxb[r]], sem_s[r]).wait()

        plsc.subcore_barrier()

        @pl.when(c == 0)
        def _():
            pltpu.sync_copy(acc.at[pl.ds(s * rpt, rpt)],
                            out0_hbm.at[pl.ds(s * rpt, rpt)])

        @pl.when(c == 1)
        def _():
            pltpu.sync_copy(acc.at[pl.ds(s * rpt, rpt)],
                            out1_hbm.at[pl.ds(s * rpt, rpt)])

    return k(flat, zeros_c, ones)


def _rel_matmul(c0, c1, rel):
    nn, nr = c0.shape
    blk = 1000
    assert nn % blk == 0

    def body(c0_ref, c1_ref, r_ref, o_ref):
        cm = c0_ref[...] + c1_ref[...]
        o_ref[...] = jnp.dot(cm, r_ref[...], preferred_element_type=jnp.float32)

    return pl.pallas_call(
        body,
        grid=(nn // blk,),
        in_specs=[pl.BlockSpec((blk, nr), lambda i: (i, 0)),
                  pl.BlockSpec((blk, nr), lambda i: (i, 0)),
                  pl.BlockSpec((nr, _RANK), lambda i: (0, 0))],
        out_specs=pl.BlockSpec((blk, _RANK), lambda i: (i, 0)),
        out_shape=jax.ShapeDtypeStruct((nn, _RANK), jnp.float32),
    )(c0, c1, rel)


def _fused_layer(pair, extra, w, norm):
    nn = extra.shape[0]
    blk = 1000
    assert nn % blk == 0

    def body(p0_ref, p1_ref, e_ref, w_ref, norm_ref, o_ref):
        x = p0_ref[0] + p1_ref[0] + e_ref[...]
        y = jnp.dot(x, w_ref[...], preferred_element_type=jnp.float32)
        y = y * norm_ref[...]
        o_ref[...] = jnp.where(y >= 0, y, y * _SLOPE)

    in_specs = [pl.BlockSpec((1, blk, _RANK), lambda i: (0, i, 0)),
                pl.BlockSpec((1, blk, _RANK), lambda i: (1, i, 0)),
                pl.BlockSpec((blk, _RANK), lambda i: (i, 0)),
                pl.BlockSpec((_RANK, _RANK), lambda i: (0, 0)),
                pl.BlockSpec((blk, 1), lambda i: (i, 0))]
    return pl.pallas_call(
        body,
        grid=(nn // blk,),
        in_specs=in_specs,
        out_specs=pl.BlockSpec((blk, _RANK), lambda i: (i, 0)),
        out_shape=jax.ShapeDtypeStruct((nn, _RANK), jnp.float32),
    )(pair, pair, extra, w, norm)


def kernel(ent_emb, rel_emb, norm, edge_index, edge_type, W0, W1):
    src = edge_index[0].astype(jnp.int32)
    dst = edge_index[1].astype(jnp.int32)
    et = edge_type.astype(jnp.int32)
    n_nodes = ent_emb.shape[0]
    nr = rel_emb.shape[0]
    n_rows = n_nodes * nr
    n_pad = -(-n_rows // (128 * _NS)) * (128 * _NS)
    zeros = jnp.zeros((_ACC_ROWS // _NS, _RANK), jnp.float32)
    zeros_c = jnp.zeros((n_pad // _NS,), jnp.float32)
    ones = jnp.ones((_CCHUNK,), jnp.float32)

    cpart = _count(dst * nr + et, n_pad, zeros_c, ones)
    r_rel = _rel_matmul(cpart[0][:n_rows].reshape(n_nodes, nr),
                        cpart[1][:n_rows].reshape(n_nodes, nr), rel_emb)

    ne = src.shape[0]
    e_raw = ne // _NW
    e_pad = -(-e_raw // _CHUNK) * _CHUNK
    padn = e_pad - e_raw
    if padn:
        pad_idx = jnp.broadcast_to(
            jnp.arange(padn, dtype=jnp.int32) % n_nodes, (_NW, padn))
        pad_dst = jnp.broadcast_to(
            n_nodes + jnp.arange(padn, dtype=jnp.int32) % (_ACC_ROWS - n_nodes),
            (_NW, padn))
        srcp = jnp.concatenate(
            [src.reshape(_NW, e_raw), pad_idx], axis=1).reshape(-1)
        dstp = jnp.concatenate(
            [dst.reshape(_NW, e_raw), pad_dst], axis=1).reshape(-1)
    else:
        srcp, dstp = src, dst

    p = _scatter_sum(ent_emb, srcp, dstp, zeros)
    h1 = _fused_layer(p, r_rel, W0, norm)

    q = _scatter_sum(h1, srcp, dstp, zeros)
    h2 = _fused_layer(q, r_rel, W1, norm)
    return h2


# speedup vs baseline: 1.0287x; 1.0287x over previous
import functools

import jax
import jax.numpy as jnp
from jax import lax
from jax.experimental import pallas as pl
from jax.experimental.pallas import tpu as pltpu
from jax.experimental.pallas import tpu_sc as plsc

_NC = 2
_NS = 16
_NW = _NC * _NS
_RANK = 128
_ACC_ROWS = 10240
_CHUNK = 80
_CCHUNK = 80
_R = 7
_D = 5
_SR = 4
_SG = 1
_SLOPE = (1.0 / 8.0 + 1.0 / 3.0) / 2.0


def _scatter_sum(table, idx, dst, zeros):
    ne = idx.shape[0]
    e_pt = ne // _NW
    nch = e_pt // _CHUNK
    ngroups = nch // _SG
    assert e_pt * _NW == ne and ngroups * _SG * _CHUNK == e_pt
    assert ngroups >= _SR
    rpt = _ACC_ROWS // _NS
    assert zeros.shape[0] == rpt

    mesh = plsc.VectorSubcoreMesh(core_axis_name="c", subcore_axis_name="s")

    nb = _SR * _SG
    scratch = ([pltpu.VMEM((_CHUNK, _RANK), jnp.float32)] * nb
               + [pltpu.VMEM((_CHUNK,), jnp.int32)] * (2 * nb)
               + [pltpu.VMEM_SHARED((_ACC_ROWS, _RANK), jnp.float32)]
               + [pltpu.SemaphoreType.DMA] * (3 * _SR))

    @functools.partial(
        pl.kernel,
        out_type=jax.ShapeDtypeStruct((_NC, _ACC_ROWS, _RANK), jnp.float32),
        mesh=mesh,
        scratch_types=scratch,
    )
    def k(table_hbm, idx_hbm, dst_hbm, z_hbm, out_hbm, *refs):
        rows = [refs[r * _SG:(r + 1) * _SG] for r in range(_SR)]
        idxb = [refs[nb + r * _SG:nb + (r + 1) * _SG] for r in range(_SR)]
        dstb = [refs[2 * nb + r * _SG:2 * nb + (r + 1) * _SG]
                for r in range(_SR)]
        acc = refs[3 * nb]
        sem_i = refs[3 * nb + 1:3 * nb + 1 + _SR]
        sem_g = refs[3 * nb + 1 + _SR:3 * nb + 1 + 2 * _SR]
        sem_s = refs[3 * nb + 1 + 2 * _SR:3 * nb + 1 + 3 * _SR]

        c = lax.axis_index("c")
        s = lax.axis_index("s")
        w = c * _NS + s
        eb = w * e_pt

        pltpu.sync_copy(z_hbm, acc.at[pl.ds(s * rpt, rpt), :])
        plsc.subcore_barrier()

        for g in (0, 1):
            for j in range(_SG):
                b = eb + (g * _SG + j) * _CHUNK
                pltpu.sync_copy(idx_hbm.at[pl.ds(b, _CHUNK)], idxb[g][j])
                pltpu.sync_copy(dst_hbm.at[pl.ds(b, _CHUNK)], dstb[g][j])
        for j in range(_SG):
            pltpu.async_copy(table_hbm.at[idxb[0][j]], rows[0][j], sem_g[0])

        def group(t, r):
            sc = (r + 2) % _SR
            g1 = (r + 1) % _SR

            @pl.when(t >= 2)
            def _():
                for j in range(_SG):
                    pltpu.make_async_copy(rows[sc][j], acc.at[dstb[sc][j]],
                                          sem_s[sc]).wait()
            @pl.when(t + 2 < ngroups)
            def _():
                for j in range(_SG):
                    b2 = eb + ((t + 2) * _SG + j) * _CHUNK
                    pltpu.async_copy(idx_hbm.at[pl.ds(b2, _CHUNK)],
                                     idxb[sc][j], sem_i[sc])
                    pltpu.async_copy(dst_hbm.at[pl.ds(b2, _CHUNK)],
                                     dstb[sc][j], sem_i[sc])
            @pl.when(t + 1 < ngroups)
            def _():
                @pl.when(t >= 1)
                def _():
                    for j in range(_SG):
                        b1 = eb + ((t + 1) * _SG + j) * _CHUNK
                        pltpu.make_async_copy(idx_hbm.at[pl.ds(b1, _CHUNK)],
                                              idxb[g1][j], sem_i[g1]).wait()
                        pltpu.make_async_copy(dst_hbm.at[pl.ds(b1, _CHUNK)],
                                              dstb[g1][j], sem_i[g1]).wait()
                for j in range(_SG):
                    pltpu.async_copy(table_hbm.at[idxb[g1][j]], rows[g1][j],
                                     sem_g[g1])
            for j in range(_SG):
                pltpu.make_async_copy(table_hbm.at[idxb[r][j]], rows[r][j],
                                      sem_g[r]).wait()
            for j in range(_SG):
                pltpu.async_copy(rows[r][j], acc.at[dstb[r][j]], sem_s[r],
                                 add=True)

        def body(t, carry):
            for r in range(_SR):
                @pl.when(lax.rem(t, _SR) == r)
                def _(r=r):
                    group(t, r)
            return carry

        lax.fori_loop(0, ngroups, body, 0)
        for t in range(ngroups - 2, ngroups):
            r = t % _SR
            for j in range(_SG):
                pltpu.make_async_copy(rows[r][j], acc.at[dstb[r][j]],
                                      sem_s[r]).wait()

        plsc.subcore_barrier()
        pltpu.sync_copy(acc.at[pl.ds(s * rpt, rpt), :],
                        out_hbm.at[c, pl.ds(s * rpt, rpt), :])

    return k(table, idx, dst, zeros)


def _count(flat, n_rows, zeros_c, ones):
    ne = flat.shape[0]
    e_pt = ne // _NW
    ngroups = e_pt // _CCHUNK
    assert ngroups * _CCHUNK == e_pt and ngroups >= _R
    rpt = n_rows // _NS
    assert rpt * _NS == n_rows and rpt % 128 == 0
    assert zeros_c.shape[0] == rpt

    mesh = plsc.VectorSubcoreMesh(core_axis_name="c", subcore_axis_name="s")

    scratch = ([pltpu.VMEM((_CCHUNK,), jnp.int32)] * _R
               + [pltpu.VMEM((_CCHUNK,), jnp.float32)]
               + [pltpu.VMEM_SHARED((n_rows,), jnp.float32)]
               + [pltpu.SemaphoreType.DMA] * (2 * _R))

    @functools.partial(
        pl.kernel,
        out_type=[jax.ShapeDtypeStruct((n_rows,), jnp.float32),
                  jax.ShapeDtypeStruct((n_rows,), jnp.float32)],
        mesh=mesh,
        scratch_types=scratch,
    )
    def k(flat_hbm, z_hbm, ones_hbm, out0_hbm, out1_hbm, *refs):
        idxb = refs[0:_R]
        ones_v = refs[_R]
        acc = refs[_R + 1]
        sem_i = refs[_R + 2:2 * _R + 2]
        sem_s = refs[2 * _R + 2:3 * _R + 2]
        c = lax.axis_index("c")
        s = lax.axis_index("s")
        w = c * _NS + s
        eb = w * e_pt

        pltpu.sync_copy(z_hbm, acc.at[pl.ds(s * rpt, rpt)])
        pltpu.sync_copy(ones_hbm, ones_v)
        plsc.subcore_barrier()

        for g in (0, 1):
            pltpu.sync_copy(flat_hbm.at[pl.ds(eb + g * _CCHUNK, _CCHUNK)],
                            idxb[g])

        def group(t, r):
            sc = (r + 2) % _R

            @pl.when(t >= _D)
            def _():
                pltpu.make_async_copy(ones_v, acc.at[idxb[sc]],
                                      sem_s[sc]).wait()

            @pl.when(t + 2 < ngroups)
            def _():
                pltpu.async_copy(
                    flat_hbm.at[pl.ds(eb + (t + 2) * _CCHUNK, _CCHUNK)],
                    idxb[sc], sem_i[sc])

            @pl.when(t >= 2)
            def _():
                pltpu.make_async_copy(
                    flat_hbm.at[pl.ds(eb + t * _CCHUNK, _CCHUNK)],
                    idxb[r], sem_i[r]).wait()

            pltpu.async_copy(ones_v, acc.at[idxb[r]], sem_s[r], add=True)

        def body(t, carry):
            for r in range(_R):
                @pl.when(lax.rem(t, _R) == r)
                def _(r=r):
                    group(t, r)
            return carry

        lax.fori_loop(0, ngroups, body, 0)
        for t in range(ngroups - _D, ngroups):
            r = t % _R
            pltpu.make_async_copy(ones_v, acc.at[idxb[r]], sem_s[r]).wait()

        plsc.subcore_barrier()

        @pl.when(c == 0)
        def _():
            pltpu.sync_copy(acc.at[pl.ds(s * rpt, rpt)],
                            out0_hbm.at[pl.ds(s * rpt, rpt)])

        @pl.when(c == 1)
        def _():
            pltpu.sync_copy(acc.at[pl.ds(s * rpt, rpt)],
                            out1_hbm.at[pl.ds(s * rpt, rpt)])

    return k(flat, zeros_c, ones)


def _rel_matmul(c0, c1, rel):
    nn, nr = c0.shape
    blk = 1000
    assert nn % blk == 0

    def body(c0_ref, c1_ref, r_ref, o_ref):
        cm = c0_ref[...] + c1_ref[...]
        o_ref[...] = jnp.dot(cm, r_ref[...], preferred_element_type=jnp.float32)

    return pl.pallas_call(
        body,
        grid=(nn // blk,),
        in_specs=[pl.BlockSpec((blk, nr), lambda i: (i, 0)),
                  pl.BlockSpec((blk, nr), lambda i: (i, 0)),
                  pl.BlockSpec((nr, _RANK), lambda i: (0, 0))],
        out_specs=pl.BlockSpec((blk, _RANK), lambda i: (i, 0)),
        out_shape=jax.ShapeDtypeStruct((nn, _RANK), jnp.float32),
    )(c0, c1, rel)


def _fused_layer(pair, extra, w, norm):
    nn = extra.shape[0]
    blk = 1000
    assert nn % blk == 0

    def body(p0_ref, p1_ref, e_ref, w_ref, norm_ref, o_ref):
        x = p0_ref[0] + p1_ref[0] + e_ref[...]
        y = jnp.dot(x, w_ref[...], preferred_element_type=jnp.float32)
        y = y * norm_ref[...]
        o_ref[...] = jnp.where(y >= 0, y, y * _SLOPE)

    in_specs = [pl.BlockSpec((1, blk, _RANK), lambda i: (0, i, 0)),
                pl.BlockSpec((1, blk, _RANK), lambda i: (1, i, 0)),
                pl.BlockSpec((blk, _RANK), lambda i: (i, 0)),
                pl.BlockSpec((_RANK, _RANK), lambda i: (0, 0)),
                pl.BlockSpec((blk, 1), lambda i: (i, 0))]
    return pl.pallas_call(
        body,
        grid=(nn // blk,),
        in_specs=in_specs,
        out_specs=pl.BlockSpec((blk, _RANK), lambda i: (i, 0)),
        out_shape=jax.ShapeDtypeStruct((nn, _RANK), jnp.float32),
    )(pair, pair, extra, w, norm)


def kernel(ent_emb, rel_emb, norm, edge_index, edge_type, W0, W1):
    src = edge_index[0].astype(jnp.int32)
    dst = edge_index[1].astype(jnp.int32)
    et = edge_type.astype(jnp.int32)
    n_nodes = ent_emb.shape[0]
    nr = rel_emb.shape[0]
    n_rows = n_nodes * nr
    n_pad = -(-n_rows // (128 * _NS)) * (128 * _NS)
    zeros = jnp.zeros((_ACC_ROWS // _NS, _RANK), jnp.float32)
    zeros_c = jnp.zeros((n_pad // _NS,), jnp.float32)
    ones = jnp.ones((_CCHUNK,), jnp.float32)

    cpart = _count(dst * nr + et, n_pad, zeros_c, ones)
    r_rel = _rel_matmul(cpart[0][:n_rows].reshape(n_nodes, nr),
                        cpart[1][:n_rows].reshape(n_nodes, nr), rel_emb)

    p = _scatter_sum(ent_emb, src, dst, zeros)
    h1 = _fused_layer(p, r_rel, W0, norm)

    q = _scatter_sum(h1, src, dst, zeros)
    h2 = _fused_layer(q, r_rel, W1, norm)
    return h2


# speedup vs baseline: 1.0411x; 1.0121x over previous
import functools

import jax
import jax.numpy as jnp
from jax import lax
from jax.experimental import pallas as pl
from jax.experimental.pallas import tpu as pltpu
from jax.experimental.pallas import tpu_sc as plsc

_NC = 2
_NS = 16
_NW = _NC * _NS
_RANK = 128
_ACC_ROWS = 10240
_CHUNK = 80
_CCHUNK = 128
_R = 7
_D = 5
_SR = 4
_SG = 1
_SLOPE = (1.0 / 8.0 + 1.0 / 3.0) / 2.0


def _scatter_sum(table, idx, dst, zeros):
    ne = idx.shape[0]
    e_pt = ne // _NW
    nch = e_pt // _CHUNK
    ngroups = nch // _SG
    assert e_pt * _NW == ne and ngroups * _SG * _CHUNK == e_pt
    assert ngroups >= _SR
    rpt = _ACC_ROWS // _NS
    assert zeros.shape[0] == rpt

    mesh = plsc.VectorSubcoreMesh(core_axis_name="c", subcore_axis_name="s")

    nb = _SR * _SG
    scratch = ([pltpu.VMEM((_CHUNK, _RANK), jnp.float32)] * nb
               + [pltpu.VMEM((_CHUNK,), jnp.int32)] * (2 * nb)
               + [pltpu.VMEM_SHARED((_ACC_ROWS, _RANK), jnp.float32)]
               + [pltpu.SemaphoreType.DMA] * (3 * _SR))

    @functools.partial(
        pl.kernel,
        out_type=jax.ShapeDtypeStruct((_NC, _ACC_ROWS, _RANK), jnp.float32),
        mesh=mesh,
        scratch_types=scratch,
    )
    def k(table_hbm, idx_hbm, dst_hbm, z_hbm, out_hbm, *refs):
        rows = [refs[r * _SG:(r + 1) * _SG] for r in range(_SR)]
        idxb = [refs[nb + r * _SG:nb + (r + 1) * _SG] for r in range(_SR)]
        dstb = [refs[2 * nb + r * _SG:2 * nb + (r + 1) * _SG]
                for r in range(_SR)]
        acc = refs[3 * nb]
        sem_i = refs[3 * nb + 1:3 * nb + 1 + _SR]
        sem_g = refs[3 * nb + 1 + _SR:3 * nb + 1 + 2 * _SR]
        sem_s = refs[3 * nb + 1 + 2 * _SR:3 * nb + 1 + 3 * _SR]

        c = lax.axis_index("c")
        s = lax.axis_index("s")
        w = c * _NS + s
        eb = w * e_pt

        pltpu.sync_copy(z_hbm, acc.at[pl.ds(s * rpt, rpt), :])
        plsc.subcore_barrier()

        for g in (0, 1):
            for j in range(_SG):
                b = eb + (g * _SG + j) * _CHUNK
                pltpu.sync_copy(idx_hbm.at[pl.ds(b, _CHUNK)], idxb[g][j])
                pltpu.sync_copy(dst_hbm.at[pl.ds(b, _CHUNK)], dstb[g][j])
        for j in range(_SG):
            pltpu.async_copy(table_hbm.at[idxb[0][j]], rows[0][j], sem_g[0])

        def group(t, r):
            sc = (r + 2) % _SR
            g1 = (r + 1) % _SR

            @pl.when(t >= 2)
            def _():
                for j in range(_SG):
                    pltpu.make_async_copy(rows[sc][j], acc.at[dstb[sc][j]],
                                          sem_s[sc]).wait()
            @pl.when(t + 2 < ngroups)
            def _():
                for j in range(_SG):
                    b2 = eb + ((t + 2) * _SG + j) * _CHUNK
                    pltpu.async_copy(idx_hbm.at[pl.ds(b2, _CHUNK)],
                                     idxb[sc][j], sem_i[sc])
                    pltpu.async_copy(dst_hbm.at[pl.ds(b2, _CHUNK)],
                                     dstb[sc][j], sem_i[sc])
            @pl.when(t + 1 < ngroups)
            def _():
                @pl.when(t >= 1)
                def _():
                    for j in range(_SG):
                        b1 = eb + ((t + 1) * _SG + j) * _CHUNK
                        pltpu.make_async_copy(idx_hbm.at[pl.ds(b1, _CHUNK)],
                                              idxb[g1][j], sem_i[g1]).wait()
                        pltpu.make_async_copy(dst_hbm.at[pl.ds(b1, _CHUNK)],
                                              dstb[g1][j], sem_i[g1]).wait()
                for j in range(_SG):
                    pltpu.async_copy(table_hbm.at[idxb[g1][j]], rows[g1][j],
                                     sem_g[g1])
            for j in range(_SG):
                pltpu.make_async_copy(table_hbm.at[idxb[r][j]], rows[r][j],
                                      sem_g[r]).wait()
            for j in range(_SG):
                pltpu.async_copy(rows[r][j], acc.at[dstb[r][j]], sem_s[r],
                                 add=True)

        def body(t, carry):
            for r in range(_SR):
                @pl.when(lax.rem(t, _SR) == r)
                def _(r=r):
                    group(t, r)
            return carry

        lax.fori_loop(0, ngroups, body, 0)
        for t in range(ngroups - 2, ngroups):
            r = t % _SR
            for j in range(_SG):
                pltpu.make_async_copy(rows[r][j], acc.at[dstb[r][j]],
                                      sem_s[r]).wait()

        plsc.subcore_barrier()
        pltpu.sync_copy(acc.at[pl.ds(s * rpt, rpt), :],
                        out_hbm.at[c, pl.ds(s * rpt, rpt), :])

    return k(table, idx, dst, zeros)


def _count(flat, n_rows, zeros_c, ones):
    ne = flat.shape[0]
    e_pt = ne // _NW
    ngroups = e_pt // _CCHUNK
    assert ngroups * _CCHUNK == e_pt and ngroups >= _R
    rpt = n_rows // _NS
    assert rpt * _NS == n_rows and rpt % 128 == 0
    assert zeros_c.shape[0] == rpt

    mesh = plsc.VectorSubcoreMesh(core_axis_name="c", subcore_axis_name="s")

    scratch = ([pltpu.VMEM((_CCHUNK,), jnp.int32)] * _R
               + [pltpu.VMEM((_CCHUNK,), jnp.float32)]
               + [pltpu.VMEM_SHARED((n_rows,), jnp.float32)]
               + [pltpu.SemaphoreType.DMA] * (2 * _R))

    @functools.partial(
        pl.kernel,
        out_type=[jax.ShapeDtypeStruct((n_rows,), jnp.float32),
                  jax.ShapeDtypeStruct((n_rows,), jnp.float32)],
        mesh=mesh,
        scratch_types=scratch,
    )
    def k(flat_hbm, z_hbm, ones_hbm, out0_hbm, out1_hbm, *refs):
        idxb = refs[0:_R]
        ones_v = refs[_R]
        acc = refs[_R + 1]
        sem_i = refs[_R + 2:2 * _R + 2]
        sem_s = refs[2 * _R + 2:3 * _R + 2]
        c = lax.axis_index("c")
        s = lax.axis_index("s")
        w = c * _NS + s
        eb = w * e_pt

        pltpu.sync_copy(z_hbm, acc.at[pl.ds(s * rpt, rpt)])
        pltpu.sync_copy(ones_hbm, ones_v)
        plsc.subcore_barrier()

        for g in (0, 1):
            pltpu.sync_copy(flat_hbm.at[pl.ds(eb + g * _CCHUNK, _CCHUNK)],
                            idxb[g])

        def group(t, r):
            sc = (r + 2) % _R

            @pl.when(t >= _D)
            def _():
                pltpu.make_async_copy(ones_v, acc.at[idxb[sc]],
                                      sem_s[sc]).wait()

            @pl.when(t + 2 < ngroups)
            def _():
                pltpu.async_copy(
                    flat_hbm.at[pl.ds(eb + (t + 2) * _CCHUNK, _CCHUNK)],
                    idxb[sc], sem_i[sc])

            @pl.when(t >= 2)
            def _():
                pltpu.make_async_copy(
                    flat_hbm.at[pl.ds(eb + t * _CCHUNK, _CCHUNK)],
                    idxb[r], sem_i[r]).wait()

            pltpu.async_copy(ones_v, acc.at[idxb[r]], sem_s[r], add=True)

        def body(t, carry):
            for r in range(_R):
                @pl.when(lax.rem(t, _R) == r)
                def _(r=r):
                    group(t, r)
            return carry

        lax.fori_loop(0, ngroups, body, 0)
        for t in range(ngroups - _D, ngroups):
            r = t % _R
            pltpu.make_async_copy(ones_v, acc.at[idxb[r]], sem_s[r]).wait()

        plsc.subcore_barrier()

        @pl.when(c == 0)
        def _():
            pltpu.sync_copy(acc.at[pl.ds(s * rpt, rpt)],
                            out0_hbm.at[pl.ds(s * rpt, rpt)])

        @pl.when(c == 1)
        def _():
            pltpu.sync_copy(acc.at[pl.ds(s * rpt, rpt)],
                            out1_hbm.at[pl.ds(s * rpt, rpt)])

    return k(flat, zeros_c, ones)


def _rel_matmul(c0, c1, rel):
    nn, nr = c0.shape
    blk = 1000
    assert nn % blk == 0

    def body(c0_ref, c1_ref, r_ref, o_ref):
        cm = c0_ref[...] + c1_ref[...]
        o_ref[...] = jnp.dot(cm, r_ref[...], preferred_element_type=jnp.float32)

    return pl.pallas_call(
        body,
        grid=(nn // blk,),
        in_specs=[pl.BlockSpec((blk, nr), lambda i: (i, 0)),
                  pl.BlockSpec((blk, nr), lambda i: (i, 0)),
                  pl.BlockSpec((nr, _RANK), lambda i: (0, 0))],
        out_specs=pl.BlockSpec((blk, _RANK), lambda i: (i, 0)),
        out_shape=jax.ShapeDtypeStruct((nn, _RANK), jnp.float32),
    )(c0, c1, rel)


def _fused_layer(pair, extra, w, norm):
    nn = extra.shape[0]
    blk = 1000
    assert nn % blk == 0

    def body(p0_ref, p1_ref, e_ref, w_ref, norm_ref, o_ref):
        x = p0_ref[0] + p1_ref[0] + e_ref[...]
        y = jnp.dot(x, w_ref[...], preferred_element_type=jnp.float32)
        y = y * norm_ref[...]
        o_ref[...] = jnp.where(y >= 0, y, y * _SLOPE)

    in_specs = [pl.BlockSpec((1, blk, _RANK), lambda i: (0, i, 0)),
                pl.BlockSpec((1, blk, _RANK), lambda i: (1, i, 0)),
                pl.BlockSpec((blk, _RANK), lambda i: (i, 0)),
                pl.BlockSpec((_RANK, _RANK), lambda i: (0, 0)),
                pl.BlockSpec((blk, 1), lambda i: (i, 0))]
    return pl.pallas_call(
        body,
        grid=(nn // blk,),
        in_specs=in_specs,
        out_specs=pl.BlockSpec((blk, _RANK), lambda i: (i, 0)),
        out_shape=jax.ShapeDtypeStruct((nn, _RANK), jnp.float32),
    )(pair, pair, extra, w, norm)


def kernel(ent_emb, rel_emb, norm, edge_index, edge_type, W0, W1):
    src = edge_index[0].astype(jnp.int32)
    dst = edge_index[1].astype(jnp.int32)
    et = edge_type.astype(jnp.int32)
    n_nodes = ent_emb.shape[0]
    nr = rel_emb.shape[0]
    n_rows = n_nodes * nr
    n_pad = -(-n_rows // (128 * _NS)) * (128 * _NS)
    if n_pad == n_rows:
        n_pad += 128 * _NS
    zeros = jnp.zeros((_ACC_ROWS // _NS, _RANK), jnp.float32)
    zeros_c = jnp.zeros((n_pad // _NS,), jnp.float32)
    ones = jnp.ones((_CCHUNK,), jnp.float32)

    flat = dst * nr + et
    ne = flat.shape[0]
    ec_raw = ne // _NW
    ec_pad = -(-ec_raw // _CCHUNK) * _CCHUNK
    cpadn = ec_pad - ec_raw
    if cpadn:
        pad_flat = jnp.broadcast_to(
            n_rows + jnp.arange(cpadn, dtype=jnp.int32) % (n_pad - n_rows),
            (_NW, cpadn))
        flat = jnp.concatenate(
            [flat.reshape(_NW, ec_raw), pad_flat], axis=1).reshape(-1)
    cpart = _count(flat, n_pad, zeros_c, ones)
    r_rel = _rel_matmul(cpart[0][:n_rows].reshape(n_nodes, nr),
                        cpart[1][:n_rows].reshape(n_nodes, nr), rel_emb)

    p = _scatter_sum(ent_emb, src, dst, zeros)
    h1 = _fused_layer(p, r_rel, W0, norm)

    q = _scatter_sum(h1, src, dst, zeros)
    h2 = _fused_layer(q, r_rel, W1, norm)
    return h2
